# baseline (speedup 1.0000x reference)
import functools

import jax
import jax.numpy as jnp
from jax import lax
from jax.experimental import pallas as pl
from jax.experimental.pallas import tpu as pltpu
from jax.experimental.pallas import tpu_sc as plsc

SIGMA_ = 1.0
CP_ = 16

NC_ = 2
NS_ = 16
NW_ = NC_ * NS_
N_ = 131072
D_ = 128
ROWS_PER_W_ = N_ // NW_
CHUNK_ = 128
NCHUNK_ = ROWS_PER_W_ // CHUNK_


def _sc_stats_body(mu_hbm, lab_hbm, sums_out, sqs_out, cnt_out,
                   idx_chunk, rows, sq, onesb, tmp, tmpc,
                   acc_sums, acc_sqs, acc_cnt):
    cid = lax.axis_index("c")
    sid = lax.axis_index("s")
    wid = cid * NS_ + sid

    def ones_body(r, _):
        for jj in range(D_ // 16):
            onesb[r, pl.ds(jj * 16, 16)] = jnp.ones((16,), jnp.float32)
        return 0
    lax.fori_loop(0, CHUNK_, ones_body, 0)

    @pl.when(sid == 0)
    def _():
        def z_body(r, _):
            for jj in range(D_ // 16):
                tmp[r, pl.ds(jj * 16, 16)] = jnp.zeros((16,), jnp.float32)
            for jj in range(D_ // 16):
                tmpc[r, pl.ds(jj * 16, 16)] = jnp.zeros((16,), jnp.float32)
            return 0
        lax.fori_loop(0, CP_, z_body, 0)
        pltpu.sync_copy(tmp, acc_sums)
        pltpu.sync_copy(tmp, acc_sqs)
        pltpu.sync_copy(tmpc, acc_cnt)

    plsc.subcore_barrier()

    def chunk_body(j, _):
        base = wid * ROWS_PER_W_ + j * CHUNK_
        pltpu.sync_copy(lab_hbm.at[pl.ds(base, CHUNK_)], idx_chunk)
        pltpu.sync_copy(mu_hbm.at[pl.ds(base, CHUNK_)], rows)

        def row_body(r, _):
            for jj in range(D_ // 16):
                v = rows[r, pl.ds(jj * 16, 16)]
                sq[r, pl.ds(jj * 16, 16)] = v * v
            return 0
        lax.fori_loop(0, CHUNK_, row_body, 0)

        pltpu.sync_copy(rows, acc_sums.at[idx_chunk], add=True)
        pltpu.sync_copy(sq, acc_sqs.at[idx_chunk], add=True)
        pltpu.sync_copy(onesb, acc_cnt.at[idx_chunk], add=True)
        return 0
    lax.fori_loop(0, NCHUNK_, chunk_body, 0)

    plsc.subcore_barrier()

    @pl.when(sid == 0)
    def _():
        pltpu.sync_copy(acc_sums, tmp)
        pltpu.sync_copy(tmp, sums_out.at[cid])
        pltpu.sync_copy(acc_sqs, tmp)
        pltpu.sync_copy(tmp, sqs_out.at[cid])
        pltpu.sync_copy(acc_cnt, tmpc)
        pltpu.sync_copy(tmpc, cnt_out.at[cid])


_sc_stats = functools.partial(
    pl.kernel,
    out_type=[
        jax.ShapeDtypeStruct((NC_, CP_, D_), jnp.float32),
        jax.ShapeDtypeStruct((NC_, CP_, D_), jnp.float32),
        jax.ShapeDtypeStruct((NC_, CP_, D_), jnp.float32),
    ],
    mesh=plsc.VectorSubcoreMesh(
        core_axis_name="c", subcore_axis_name="s",
        num_cores=NC_, num_subcores=NS_),
    scratch_types=[
        pltpu.VMEM((CHUNK_,), jnp.int32),
        pltpu.VMEM((CHUNK_, D_), jnp.float32),
        pltpu.VMEM((CHUNK_, D_), jnp.float32),
        pltpu.VMEM((CHUNK_, D_), jnp.float32),
        pltpu.VMEM((CP_, D_), jnp.float32),
        pltpu.VMEM((CP_, D_), jnp.float32),
        pltpu.VMEM_SHARED((CP_, D_), jnp.float32),
        pltpu.VMEM_SHARED((CP_, D_), jnp.float32),
        pltpu.VMEM_SHARED((CP_, D_), jnp.float32),
    ],
)(_sc_stats_body)


def _finalize_body(s_ref, q_ref, c_ref, out_ref):
    HI = lax.Precision.HIGHEST
    f32 = jnp.float32
    x = s_ref[...]
    sums = x[0:CP_] + x[CP_:2 * CP_]
    y = q_ref[...]
    sqs = y[0:CP_] + y[CP_:2 * CP_]
    z = c_ref[...]
    c2 = z[0:CP_] + z[CP_:2 * CP_]
    onesr0 = jnp.ones((1, D_), f32)
    dnT = (((1,), (1,)), ((), ()))
    counts = lax.dot_general(c2, onesr0, dnT,
                             preferred_element_type=f32, precision=HI) / float(D_)

    safe = jnp.maximum(counts, 1.0)
    muF = sums / safe
    SigF = sqs / safe - muF * muF + SIGMA_
    SigF = jnp.maximum(SigF, 1e-6)

    logS = jnp.log(SigF)
    R2 = 1.0 / SigF
    onesr = jnp.ones((1, D_), f32)
    ld_i = lax.dot_general(logS, onesr, dnT, preferred_element_type=f32, precision=HI)
    ld_j = lax.dot_general(onesr, logS, dnT, preferred_element_type=f32, precision=HI)
    A = lax.dot_general(SigF, R2, dnT, preferred_element_type=f32, precision=HI)
    m2 = muF * muF
    B1 = lax.dot_general(m2, R2, dnT, preferred_element_type=f32, precision=HI)
    B2 = lax.dot_general(muF, muF * R2, dnT, preferred_element_type=f32, precision=HI)
    t_j = lax.dot_general(onesr, m2 * R2, dnT, preferred_element_type=f32, precision=HI)

    D = float(D_)
    kl = 0.5 * (ld_j - ld_i + A + B1 - 2.0 * B2 + t_j - D)

    pres_i = (counts > 0.0).astype(f32)
    ones11 = jnp.ones((1, 1), f32)
    pres_j = lax.dot_general(ones11, pres_i, dnT, preferred_element_type=f32, precision=HI)
    ri = lax.broadcasted_iota(jnp.int32, (CP_, CP_), 0)
    ci = lax.broadcasted_iota(jnp.int32, (CP_, CP_), 1)
    off_diag = (ri != ci).astype(f32)
    mask = pres_i * pres_j * off_diag
    key_num = jnp.sum(pres_i)
    denom = jnp.maximum(key_num * (key_num - 1.0), 1.0)
    loss = jnp.sum(kl * mask) / denom
    out_ref[...] = jnp.broadcast_to(loss, (1, 1))


def kernel(mu, private_label):
    labels = private_label.astype(jnp.int32)
    sums2, sqs2, cnt2 = _sc_stats(mu, labels)
    loss = pl.pallas_call(
        _finalize_body,
        out_shape=jax.ShapeDtypeStruct((1, 1), jnp.float32),
    )(sums2.reshape(NC_ * CP_, D_),
      sqs2.reshape(NC_ * CP_, D_),
      cnt2.reshape(NC_ * CP_, D_))
    return loss[0, 0]


# speedup vs baseline: 1.8023x; 1.8023x over previous
import functools

import jax
import jax.numpy as jnp
from jax import lax
from jax.experimental import pallas as pl
from jax.experimental.pallas import tpu as pltpu
from jax.experimental.pallas import tpu_sc as plsc

SIGMA_ = 1.0
CP_ = 16

NC_ = 2
NS_ = 16
NW_ = NC_ * NS_
N_ = 131072
D_ = 128
ROWS_PER_W_ = N_ // NW_
CHUNK_ = 128
NCHUNK_ = ROWS_PER_W_ // CHUNK_
NPAIR_ = NCHUNK_ // 2


def _sc_stats_body(mu_hbm, lab_hbm, sums_out, sqs_out,
                   idx0, idx1, rows0, rows1, sq0, sq1, tmp,
                   acc_sums, acc_sqs,
                   insem0, insem1, scsem0, scsem1):
    cid = lax.axis_index("c")
    sid = lax.axis_index("s")
    wid = cid * NS_ + sid
    base_w = wid * ROWS_PER_W_

    bufs = ((idx0, rows0, sq0, insem0, scsem0),
            (idx1, rows1, sq1, insem1, scsem1))

    def start_in(j, b):
        idx, rows, _, insem, _ = bufs[b]
        base = base_w + j * CHUNK_
        pltpu.async_copy(lab_hbm.at[pl.ds(base, CHUNK_)], idx, insem)
        pltpu.async_copy(mu_hbm.at[pl.ds(base, CHUNK_)], rows, insem)

    def wait_in(b):
        idx, rows, _, insem, _ = bufs[b]
        pltpu.make_async_copy(lab_hbm.at[pl.ds(base_w, CHUNK_)], idx, insem).wait()
        pltpu.make_async_copy(mu_hbm.at[pl.ds(base_w, CHUNK_)], rows, insem).wait()

    def squares(b):
        _, rows, sq, _, _ = bufs[b]

        def row_body(r, _):
            for jj in range(D_ // 16):
                v = rows[r, pl.ds(jj * 16, 16)]
                sq[r, pl.ds(jj * 16, 16)] = v * v
            return 0
        lax.fori_loop(0, CHUNK_, row_body, 0)

    def start_scatter(b):
        idx, rows, sq, _, scsem = bufs[b]
        pltpu.async_copy(rows, acc_sums.at[idx], scsem, add=True)
        pltpu.async_copy(sq, acc_sqs.at[idx], scsem, add=True)

    def wait_scatter(b):
        idx, rows, sq, _, scsem = bufs[b]
        pltpu.make_async_copy(rows, acc_sums.at[idx], scsem).wait()
        pltpu.make_async_copy(sq, acc_sqs.at[idx], scsem).wait()

    @pl.when(sid == 0)
    def _():
        def z_body(r, _):
            for jj in range(D_ // 16):
                tmp[r, pl.ds(jj * 16, 16)] = jnp.zeros((16,), jnp.float32)
            return 0
        lax.fori_loop(0, CP_, z_body, 0)
        pltpu.sync_copy(tmp, acc_sums)
        pltpu.sync_copy(tmp, acc_sqs)

    plsc.subcore_barrier()

    start_in(0, 0)

    def pair_body(jo, _):
        a = 2 * jo
        wait_in(0)
        squares(0)

        @pl.when(jo > 0)
        def _():
            wait_scatter(1)
        start_scatter(0)
        start_in(a + 1, 1)

        wait_in(1)
        squares(1)
        start_scatter(1)
        wait_scatter(0)

        @pl.when(jo < NPAIR_ - 1)
        def _():
            start_in(a + 2, 0)
        return 0
    lax.fori_loop(0, NPAIR_, pair_body, 0)

    wait_scatter(1)

    plsc.subcore_barrier()

    @pl.when(sid == 0)
    def _():
        pltpu.sync_copy(acc_sums, tmp)
        pltpu.sync_copy(tmp, sums_out.at[cid])
        pltpu.sync_copy(acc_sqs, tmp)
        pltpu.sync_copy(tmp, sqs_out.at[cid])


_sc_stats = functools.partial(
    pl.kernel,
    out_type=[
        jax.ShapeDtypeStruct((NC_, CP_, D_), jnp.float32),
        jax.ShapeDtypeStruct((NC_, CP_, D_), jnp.float32),
    ],
    mesh=plsc.VectorSubcoreMesh(
        core_axis_name="c", subcore_axis_name="s",
        num_cores=NC_, num_subcores=NS_),
    scratch_types=[
        pltpu.VMEM((CHUNK_,), jnp.int32),
        pltpu.VMEM((CHUNK_,), jnp.int32),
        pltpu.VMEM((CHUNK_, D_), jnp.float32),
        pltpu.VMEM((CHUNK_, D_), jnp.float32),
        pltpu.VMEM((CHUNK_, D_), jnp.float32),
        pltpu.VMEM((CHUNK_, D_), jnp.float32),
        pltpu.VMEM((CP_, D_), jnp.float32),
        pltpu.VMEM_SHARED((CP_, D_), jnp.float32),
        pltpu.VMEM_SHARED((CP_, D_), jnp.float32),
        pltpu.SemaphoreType.DMA,
        pltpu.SemaphoreType.DMA,
        pltpu.SemaphoreType.DMA,
        pltpu.SemaphoreType.DMA,
    ],
)(_sc_stats_body)


def _counts_body(lab_ref, cnt_ref):
    i = pl.program_id(0)
    B = lab_ref.shape[2]
    labs = lab_ref[0]
    labs_b = jnp.broadcast_to(labs, (CP_, B))
    cls = lax.broadcasted_iota(jnp.int32, (CP_, B), 0)
    oh = (labs_b == cls).astype(jnp.float32)
    c = jnp.sum(oh, axis=1, keepdims=True)
    cb = jnp.broadcast_to(c, (CP_, D_))

    @pl.when(i == 0)
    def _():
        cnt_ref[...] = cb

    @pl.when(i != 0)
    def _():
        cnt_ref[...] += cb


def _finalize_body(s_ref, q_ref, c_ref, out_ref):
    HI = lax.Precision.HIGHEST
    f32 = jnp.float32
    x = s_ref[...]
    sums = x[0:CP_] + x[CP_:2 * CP_]
    y = q_ref[...]
    sqs = y[0:CP_] + y[CP_:2 * CP_]
    c2 = c_ref[...]
    onesr = jnp.ones((1, D_), f32)
    dnT = (((1,), (1,)), ((), ()))
    counts = lax.dot_general(c2, onesr, dnT,
                             preferred_element_type=f32, precision=HI) / float(D_)

    safe = jnp.maximum(counts, 1.0)
    muF = sums / safe
    SigF = sqs / safe - muF * muF + SIGMA_
    SigF = jnp.maximum(SigF, 1e-6)

    logS = jnp.log(SigF)
    R2 = 1.0 / SigF
    ld_i = lax.dot_general(logS, onesr, dnT, preferred_element_type=f32, precision=HI)
    ld_j = lax.dot_general(onesr, logS, dnT, preferred_element_type=f32, precision=HI)
    A = lax.dot_general(SigF, R2, dnT, preferred_element_type=f32, precision=HI)
    m2 = muF * muF
    B1 = lax.dot_general(m2, R2, dnT, preferred_element_type=f32, precision=HI)
    B2 = lax.dot_general(muF, muF * R2, dnT, preferred_element_type=f32, precision=HI)
    t_j = lax.dot_general(onesr, m2 * R2, dnT, preferred_element_type=f32, precision=HI)

    D = float(D_)
    kl = 0.5 * (ld_j - ld_i + A + B1 - 2.0 * B2 + t_j - D)

    pres_i = (counts > 0.0).astype(f32)
    ones11 = jnp.ones((1, 1), f32)
    pres_j = lax.dot_general(ones11, pres_i, dnT, preferred_element_type=f32, precision=HI)
    ri = lax.broadcasted_iota(jnp.int32, (CP_, CP_), 0)
    ci = lax.broadcasted_iota(jnp.int32, (CP_, CP_), 1)
    off_diag = (ri != ci).astype(f32)
    mask = pres_i * pres_j * off_diag
    key_num = jnp.sum(pres_i)
    denom = jnp.maximum(key_num * (key_num - 1.0), 1.0)
    loss = jnp.sum(kl * mask) / denom
    out_ref[...] = jnp.broadcast_to(loss, (1, 1))


def kernel(mu, private_label):
    labels = private_label.astype(jnp.int32)
    sums2, sqs2 = _sc_stats(mu, labels)

    CB = 8192
    GC = N_ // CB
    cnt = pl.pallas_call(
        _counts_body,
        grid=(GC,),
        in_specs=[pl.BlockSpec((1, 1, CB), lambda i: (i, 0, 0))],
        out_specs=pl.BlockSpec((CP_, D_), lambda i: (0, 0)),
        out_shape=jax.ShapeDtypeStruct((CP_, D_), jnp.float32),
    )(labels.reshape(GC, 1, CB))

    loss = pl.pallas_call(
        _finalize_body,
        out_shape=jax.ShapeDtypeStruct((1, 1), jnp.float32),
    )(sums2.reshape(NC_ * CP_, D_),
      sqs2.reshape(NC_ * CP_, D_),
      cnt)
    return loss[0, 0]


# speedup vs baseline: 2.7206x; 1.5095x over previous
import jax
import jax.numpy as jnp
from jax import lax
from jax.experimental import pallas as pl

SIGMA_ = 1.0
C_ = 10
CP_ = 16


def _stats_body(lab_ref, mu_ref, sums_ref, sqs_ref, cnt_ref):
    i = pl.program_id(0)
    B = mu_ref.shape[0]
    labs = lab_ref[0]
    labs_b = jnp.broadcast_to(labs, (CP_, B))
    cls = lax.broadcasted_iota(jnp.int32, (CP_, B), 0)
    oh = (labs_b == cls).astype(jnp.float32)
    m = mu_ref[...]
    dn = (((1,), (0,)), ((), ()))
    ohb = oh.astype(jnp.bfloat16)
    f32 = jnp.float32

    def split_dot(x):
        hi = x.astype(jnp.bfloat16)
        lo = (x - hi.astype(f32)).astype(jnp.bfloat16)
        a = lax.dot_general(ohb, hi, dn, preferred_element_type=f32)
        b = lax.dot_general(ohb, lo, dn, preferred_element_type=f32)
        return a + b

    s = split_dot(m)
    q = split_dot(m * m)
    c = jnp.sum(oh, axis=1, keepdims=True)

    cb = jnp.broadcast_to(c, (CP_, 128))

    @pl.when(i == 0)
    def _():
        sums_ref[...] = s
        sqs_ref[...] = q
        cnt_ref[...] = cb

    @pl.when(i != 0)
    def _():
        sums_ref[...] += s
        sqs_ref[...] += q
        cnt_ref[...] += cb


def _finalize_body(sums_ref, sqs_ref, cnt_ref, out_ref):
    counts = cnt_ref[...]
    sums = sums_ref[...]
    sqs = sqs_ref[...]
    safe = jnp.maximum(counts, 1.0)
    muF = sums / safe
    SigF = sqs / safe - muF * muF + SIGMA_
    SigF = jnp.maximum(SigF, 1e-6)

    logS = jnp.log(SigF)
    R2 = 1.0 / SigF
    onesr = jnp.ones((1, 128), jnp.float32)
    dnT = (((1,), (1,)), ((), ()))
    f32 = jnp.float32
    ld_i = lax.dot_general(logS, onesr, dnT, preferred_element_type=f32, precision=lax.Precision.HIGHEST)
    ld_j = lax.dot_general(onesr, logS, dnT, preferred_element_type=f32, precision=lax.Precision.HIGHEST)
    A = lax.dot_general(SigF, R2, dnT, preferred_element_type=f32, precision=lax.Precision.HIGHEST)
    m2 = muF * muF
    B1 = lax.dot_general(m2, R2, dnT, preferred_element_type=f32, precision=lax.Precision.HIGHEST)
    B2 = lax.dot_general(muF, muF * R2, dnT, preferred_element_type=f32, precision=lax.Precision.HIGHEST)
    t_j = lax.dot_general(onesr, m2 * R2, dnT, preferred_element_type=f32, precision=lax.Precision.HIGHEST)

    D = 128.0
    kl = 0.5 * (ld_j - ld_i + A + B1 - 2.0 * B2 + t_j - D)

    pres = (counts > 0.0).astype(f32)
    pres_i = lax.dot_general(pres, onesr / D, dnT, preferred_element_type=f32, precision=lax.Precision.HIGHEST)
    pres_j = lax.dot_general(onesr / D, pres, dnT, preferred_element_type=f32, precision=lax.Precision.HIGHEST)
    ri = lax.broadcasted_iota(jnp.int32, (CP_, CP_), 0)
    ci = lax.broadcasted_iota(jnp.int32, (CP_, CP_), 1)
    off_diag = (ri != ci).astype(f32)
    mask = pres_i * pres_j * off_diag
    key_num = jnp.sum(pres_i)
    denom = jnp.maximum(key_num * (key_num - 1.0), 1.0)
    loss = jnp.sum(kl * mask) / denom
    out_ref[...] = jnp.broadcast_to(loss, (1, 1))


def _gmm_loss(mu, labels_3d):
    G = labels_3d.shape[0]
    B = labels_3d.shape[2]
    sums, sqs, cnt = pl.pallas_call(
        _stats_body,
        grid=(G,),
        in_specs=[
            pl.BlockSpec((1, 1, B), lambda i: (i, 0, 0)),
            pl.BlockSpec((B, 128), lambda i: (i, 0)),
        ],
        out_specs=[
            pl.BlockSpec((CP_, 128), lambda i: (0, 0)),
            pl.BlockSpec((CP_, 128), lambda i: (0, 0)),
            pl.BlockSpec((CP_, 128), lambda i: (0, 0)),
        ],
        out_shape=[
            jax.ShapeDtypeStruct((CP_, 128), jnp.float32),
            jax.ShapeDtypeStruct((CP_, 128), jnp.float32),
            jax.ShapeDtypeStruct((CP_, 128), jnp.float32),
        ],
    )(labels_3d, mu)

    loss = pl.pallas_call(
        _finalize_body,
        out_shape=jax.ShapeDtypeStruct((1, 1), jnp.float32),
    )(sums, sqs, cnt)
    return loss[0, 0]


def kernel(mu, private_label):
    N, D = mu.shape
    B = 2048
    G = N // B
    labels = private_label.astype(jnp.int32).reshape(G, 1, B)
    return _gmm_loss(mu, labels)
